# baseline (speedup 1.0000x reference)
import functools
import math

import jax
import jax.numpy as jnp
from jax import lax
from jax.experimental import pallas as pl
from jax.experimental.pallas import tpu as pltpu
from jax.experimental.pallas import tpu_sc as plsc

_INFO = plsc.get_sparse_core_info()
_NC = _INFO.num_cores
_NS = _INFO.num_subcores
_NW = _NC * _NS
_LANES = _INFO.num_lanes

_NSET = 4
_ROW_UNROLL = 8


def _scale_transpose(W):
    vocab, dim = W.shape
    wt = jnp.swapaxes(W, 0, 1)
    blk = 512
    grid = pl.cdiv(vocab, blk)

    def body(wt_ref, out_ref):
        out_ref[:, 0:dim] = jnp.transpose(wt_ref[...], (1, 0)) * 8.0
        out_ref[:, dim:2 * dim] = jnp.zeros((blk, dim), jnp.float32)

    return pl.pallas_call(
        body,
        grid=(grid,),
        in_specs=[pl.BlockSpec((dim, blk), lambda j: (0, j))],
        out_specs=pl.BlockSpec((blk, 2 * dim), lambda j: (j, 0)),
        out_shape=jax.ShapeDtypeStruct((vocab, 2 * dim), jnp.float32),
    )(wt)


def _segments(hist):
    segs, off = [], 0
    while off < hist:
        n = min(128, hist - off)
        segs.append((off, n))
        off += n
    assert all(o % 8 == 0 and n % 8 == 0 for o, n in segs)
    return segs


def _make_lookup(batch, hist, vocab, dim):
    mesh = plsc.VectorSubcoreMesh(core_axis_name="c", subcore_axis_name="s")
    rows_per_w = batch // _NW
    half = rows_per_w // 2
    segs = _segments(hist)

    assert rows_per_w % 4 == 0 and half % 4 == 0 and half >= 8

    scratch = [pltpu.VMEM((half, hist), jnp.int32)]
    for _ in range(_NSET):
        scratch += [pltpu.VMEM((n, 2 * dim), jnp.float32) for _, n in segs]
    scratch += [pltpu.SemaphoreType.DMA for _ in range(2 * _NSET)]

    @functools.partial(
        pl.kernel,
        mesh=mesh,
        out_type=jax.ShapeDtypeStruct((batch, hist, 2 * dim), jnp.float32),
        scratch_types=scratch,
        compiler_params=pltpu.CompilerParams(use_tc_tiling_on_sc=True),
    )
    def k(idx_hbm, table_hbm, out_hbm, idx_v, *rest):
        ns = len(segs)
        bufs = [rest[i * ns:(i + 1) * ns] for i in range(_NSET)]
        sem_g = rest[_NSET * ns:_NSET * ns + _NSET]
        sem_s = rest[_NSET * ns + _NSET:_NSET * ns + 2 * _NSET]

        wid = lax.axis_index("s") * _NC + lax.axis_index("c")
        row0 = wid * rows_per_w

        def load_idx(h):
            pltpu.sync_copy(idx_hbm.at[pl.ds(row0 + h * half, half)], idx_v)

        def issue_gathers(r, s):
            for j, (off, n) in enumerate(segs):
                pltpu.async_copy(
                    table_hbm.at[idx_v.at[r, pl.ds(off, n)]],
                    bufs[s][j], sem_g[s])

        def drain_gathers(r, s):
            for j, (off, n) in enumerate(segs):
                pltpu.make_async_copy(
                    table_hbm.at[idx_v.at[r, pl.ds(off, n)]],
                    bufs[s][j], sem_g[s]).wait()

        def issue_stores(g, s):
            for j, (off, n) in enumerate(segs):
                pltpu.async_copy(
                    bufs[s][j], out_hbm.at[row0 + g, pl.ds(off, n)], sem_s[s])

        def drain_stores(g, s):
            for j, (off, n) in enumerate(segs):
                pltpu.make_async_copy(
                    bufs[s][j], out_hbm.at[row0 + g, pl.ds(off, n)],
                    sem_s[s]).wait()

        def visit(r, g, s, drain=None, issue=None):
            drain_gathers(r, s)
            issue_stores(g, s)
            if drain is not None:
                drain_stores(drain[0], drain[1])
            if issue is not None:
                issue_gathers(issue[0], issue[2])

        load_idx(0)
        issue_gathers(0, 0)
        issue_gathers(1, 1)
        visit(0, 0, 0, None, (2, 2, 2))
        visit(1, 1, 1, None, (3, 3, 3))

        def group(p, c):
            g0 = 4 * p + 2
            for u in range(4):
                g = g0 + u
                visit(g, g, (2 + u) % _NSET, (g - 2, u % _NSET),
                      (g + 2, g + 2, u % _NSET))
            return c

        lax.fori_loop(0, (half - 4) // 4, group, 0)

        visit(half - 2, half - 2, (half - 2) % _NSET,
              (half - 4, (half - 4) % _NSET), None)
        visit(half - 1, half - 1, (half - 1) % _NSET,
              (half - 3, (half - 3) % _NSET), None)
        load_idx(1)
        issue_gathers(0, half % _NSET)
        issue_gathers(1, (half + 1) % _NSET)

        visit(0, half, 0, (half - 2, (half - 2) % _NSET), (2, half + 2, 2))
        visit(1, half + 1, 1, (half - 1, (half - 1) % _NSET),
              (3, half + 3, 3))

        def group2(p, c):
            g0 = 4 * p + 2
            for u in range(4):
                g = g0 + u
                visit(g, half + g, (2 + u) % _NSET,
                      (half + g - 2, u % _NSET),
                      (g + 2, half + g + 2, u % _NSET))
            return c

        lax.fori_loop(0, (half - 4) // 4, group2, 0)

        visit(half - 2, rows_per_w - 2, (half - 2) % _NSET,
              (rows_per_w - 4, (half - 4) % _NSET), None)
        visit(half - 1, rows_per_w - 1, (half - 1) % _NSET,
              (rows_per_w - 3, (half - 3) % _NSET), None)
        drain_stores(rows_per_w - 2, (half - 2) % _NSET)
        drain_stores(rows_per_w - 1, (half - 1) % _NSET)

    return k


def kernel(indices, W):
    batch, hist = indices.shape
    vocab, dim = W.shape
    assert batch % (2 * _NW) == 0 and dim % _LANES == 0 and 2 * dim == 128
    idx = indices if indices.dtype == jnp.int32 else indices.astype(jnp.int32)
    table = _scale_transpose(W)
    out = _make_lookup(batch, hist, vocab, dim)(idx, table)
    return out[:, :, :dim]


# speedup vs baseline: 1.6656x; 1.6656x over previous
import functools
import math

import jax
import jax.numpy as jnp
from jax import lax
from jax.experimental import pallas as pl
from jax.experimental.pallas import tpu as pltpu
from jax.experimental.pallas import tpu_sc as plsc

_INFO = plsc.get_sparse_core_info()
_NC = _INFO.num_cores
_NS = _INFO.num_subcores
_NW = _NC * _NS
_LANES = _INFO.num_lanes

_ROW_UNROLL = 8


def _segments(hist):
    segs, off = [], 0
    while off < hist:
        n = min(128, hist - off)
        segs.append((off, n))
        off += n
    assert all(o % 8 == 0 and n % 8 == 0 for o, n in segs)
    return segs


def _make_lookup(batch, hist, vocab, dim):
    mesh = plsc.VectorSubcoreMesh(core_axis_name="c", subcore_axis_name="s")
    rows_per_w = batch // _NW
    segs = _segments(hist)
    assert rows_per_w % 2 == 0 and rows_per_w >= 4

    scratch = [pltpu.VMEM((rows_per_w, hist), jnp.int32)]
    for _ in range(2):
        scratch += [pltpu.VMEM((n, 2 * dim), jnp.float32) for _, n in segs]
    scratch += [pltpu.SemaphoreType.DMA for _ in range(4)]

    @functools.partial(
        pl.kernel,
        mesh=mesh,
        out_type=jax.ShapeDtypeStruct((batch, hist, 2 * dim), jnp.float32),
        scratch_types=scratch,
        compiler_params=pltpu.CompilerParams(use_tc_tiling_on_sc=True),
    )
    def k(idx_hbm, table_hbm, out_hbm, idx_v, *rest):
        ns = len(segs)
        bufs = (rest[:ns], rest[ns:2 * ns])
        sem_g = rest[2 * ns:2 * ns + 2]
        sem_s = rest[2 * ns + 2:2 * ns + 4]

        wid = lax.axis_index("s") * _NC + lax.axis_index("c")
        row0 = wid * rows_per_w
        pltpu.sync_copy(idx_hbm.at[pl.ds(row0, rows_per_w)], idx_v)

        def issue_gathers(r, bank):
            for j, (off, n) in enumerate(segs):
                pltpu.async_copy(
                    table_hbm.at[idx_v.at[r, pl.ds(off, n)]],
                    bufs[bank][j], sem_g[bank])

        def drain_gathers(r, bank):
            for j, (off, n) in enumerate(segs):
                pltpu.make_async_copy(
                    table_hbm.at[idx_v.at[r, pl.ds(off, n)]],
                    bufs[bank][j], sem_g[bank]).wait()

        def scale(bank, j, n):
            buf = bufs[bank][j]

            def body(i, c):
                base = i * _ROW_UNROLL
                for r in range(_ROW_UNROLL):
                    for col in range(dim // _LANES):
                        sl = pl.ds(col * _LANES, _LANES)
                        buf[base + r, sl] = buf[base + r, sl] * 8.0
                return c

            lax.fori_loop(0, n // _ROW_UNROLL, body, 0)

        def issue_stores(r, bank):
            for j, (off, n) in enumerate(segs):
                scale(bank, j, n)
                pltpu.async_copy(
                    bufs[bank][j],
                    out_hbm.at[row0 + r, pl.ds(off, n)], sem_s[bank])

        def drain_stores(r, bank):
            for j, (off, n) in enumerate(segs):
                pltpu.make_async_copy(
                    bufs[bank][j],
                    out_hbm.at[row0 + r, pl.ds(off, n)], sem_s[bank]).wait()

        def visit(r, bank, reissue):
            drain_gathers(r, bank)
            issue_stores(r, bank)
            drain_stores(r, bank)
            if reissue:
                issue_gathers(r + 2, bank)

        issue_gathers(0, 0)
        issue_gathers(1, 1)

        def pair(p, c):
            visit(2 * p, 0, True)
            visit(2 * p + 1, 1, True)
            return c

        lax.fori_loop(0, rows_per_w // 2 - 1, pair, 0)

        r_last = rows_per_w - 2
        visit(r_last, 0, False)
        visit(r_last + 1, 1, False)

    return k


def kernel(indices, W):
    batch, hist = indices.shape
    vocab, dim = W.shape
    assert batch % _NW == 0 and dim % _LANES == 0 and 2 * dim == 128
    idx = indices if indices.dtype == jnp.int32 else indices.astype(jnp.int32)
    table = jnp.pad(W, ((0, 0), (0, dim)))
    out = _make_lookup(batch, hist, vocab, dim)(idx, table)
    return out[:, :, :dim]
